# baseline (speedup 1.0000x reference)
import functools

import jax
import jax.numpy as jnp
from jax import lax
from jax.experimental import pallas as pl
from jax.experimental.pallas import tpu as pltpu
from jax.experimental.pallas import tpu_sc as plsc

_B = 16384
_L = 50
_D = 32
_C = 4
_NC = 2
_NS = 16
_NW = _NC * _NS
_ROWS_W = _B // _NW
_CB = 32
_NCH = _ROWS_W // _CB
_CHUNK_IDX = _CB * _L


def _embed_bag_sum(text_flat, emb_weight):
    mesh = plsc.VectorSubcoreMesh(core_axis_name="c", subcore_axis_name="s")

    @functools.partial(
        pl.kernel,
        out_type=jax.ShapeDtypeStruct((_B, _D), jnp.float32),
        mesh=mesh,
        scratch_types=[
            pltpu.VMEM((_CB, _L), jnp.int32),
            pltpu.VMEM((_CB, _L), jnp.int32),
            pltpu.VMEM((_CB, _L, _D), jnp.float32),
            pltpu.VMEM((_CB, _L, _D), jnp.float32),
            pltpu.VMEM((_CB, _D), jnp.float32),
            pltpu.SemaphoreType.DMA,
            pltpu.SemaphoreType.DMA,
        ],
        compiler_params=pltpu.CompilerParams(use_tc_tiling_on_sc=False),
    )
    def body(text_hbm, emb_hbm, out_hbm, idx0, idx1, rows0, rows1, outb,
             sem0, sem1):
        wid = lax.axis_index("s") * _NC + lax.axis_index("c")
        base_row = wid * _ROWS_W

        idx_bufs = (idx0, idx1)
        row_bufs = (rows0, rows1)
        sems = (sem0, sem1)

        zero = jnp.zeros((16,), jnp.float32)

        def stage_and_fire(ch, b):
            pltpu.sync_copy(
                text_hbm.at[pl.ds(base_row + ch * _CB, _CB)], idx_bufs[b])

            def fire(i, _):
                pltpu.async_copy(emb_hbm.at[idx_bufs[b].at[i]],
                                 row_bufs[b].at[i], sems[b])
                return 0
            lax.fori_loop(0, _CB, fire, 0)

        stage_and_fire(0, 0)

        for ch in range(_NCH):
            cur = ch % 2
            nxt = (ch + 1) % 2
            if ch + 1 < _NCH:
                stage_and_fire(ch + 1, nxt)
            cur_rows = row_bufs[cur]
            cur_idx = idx_bufs[cur]
            cur_sem = sems[cur]

            def row_body(i, _):
                pltpu.make_async_copy(emb_hbm.at[cur_idx.at[i]],
                                      cur_rows.at[i], cur_sem).wait()

                def l_body(l, carry):
                    a0, a1 = carry
                    return (a0 + cur_rows[i, l, pl.ds(0, 16)],
                            a1 + cur_rows[i, l, pl.ds(16, 16)])
                a0, a1 = lax.fori_loop(0, _L, l_body, (zero, zero),
                                       unroll=10)
                outb[i, pl.ds(0, 16)] = a0
                outb[i, pl.ds(16, 16)] = a1
                return 0

            lax.fori_loop(0, _CB, row_body, 0)
            pltpu.sync_copy(outb, out_hbm.at[pl.ds(base_row + ch * _CB, _CB)])

    return body(text_flat, emb_weight)


def _linear_body(x_ref, w_ref, b_ref, o_ref):
    y = lax.dot_general(x_ref[...], w_ref[...], (((1,), (1,)), ((), ())),
                        preferred_element_type=jnp.float32)
    o_ref[...] = y * (1.0 / _L) + b_ref[...]


def _linear(bag, fc_weight, fc_bias2d):
    return pl.pallas_call(
        _linear_body,
        out_shape=jax.ShapeDtypeStruct((_B, _C), jnp.float32),
    )(bag, fc_weight, fc_bias2d)


def kernel(text, emb_weight, fc_weight, fc_bias):
    bag = _embed_bag_sum(text, emb_weight)
    return _linear(bag, fc_weight, fc_bias.reshape(1, _C))


# speedup vs baseline: 1.2076x; 1.2076x over previous
import functools

import jax
import jax.numpy as jnp
from jax import lax
from jax.experimental import pallas as pl
from jax.experimental.pallas import tpu as pltpu
from jax.experimental.pallas import tpu_sc as plsc

_B = 16384
_L = 50
_D = 32
_C = 4
_NC = 2
_NS = 16
_NW = _NC * _NS
_ROWS_W = _B // _NW
_CB = 32
_NCH = _ROWS_W // _CB
_CHUNK_IDX = _CB * _L


_V = 1000000
_TBLK = 8192


def _transpose_body(xt_ref, o_ref):
    lane = lax.broadcasted_iota(jnp.int32, (_D, 128), 1)
    drow = lax.broadcasted_iota(jnp.int32, (_D, 128), 0)
    rep = jnp.where(lane % _D == drow, 1.0, 0.0).astype(jnp.float32)
    y = lax.dot_general(xt_ref[...], rep, (((0,), (0,)), ((), ())),
                        preferred_element_type=jnp.float32)
    sub = lax.broadcasted_iota(jnp.int32, (_TBLK, 128), 0)
    lane2 = lax.broadcasted_iota(jnp.int32, (_TBLK, 128), 1)
    ym = jnp.where(lane2 // _D == sub % 4, y, 0.0)
    o_ref[...] = ym.reshape(_TBLK // 4, 4, 128).sum(axis=1)


def _relayout_table(emb_t):
    grid = (_V + _TBLK - 1) // _TBLK
    return pl.pallas_call(
        _transpose_body,
        grid=(grid,),
        in_specs=[pl.BlockSpec((_D, _TBLK), lambda i: (0, i))],
        out_specs=pl.BlockSpec((_TBLK * _D // 128, 128), lambda i: (i, 0)),
        out_shape=jax.ShapeDtypeStruct((_V * _D // 128, 128), jnp.float32),
    )(emb_t)


def _embed_bag_sum(text_flat, emb_weight):
    mesh = plsc.VectorSubcoreMesh(core_axis_name="c", subcore_axis_name="s")

    @functools.partial(
        pl.kernel,
        out_type=jax.ShapeDtypeStruct((_B, _D), jnp.float32),
        mesh=mesh,
        scratch_types=[
            pltpu.VMEM((_CB, _L), jnp.int32),
            pltpu.VMEM((_CB, _L), jnp.int32),
            pltpu.VMEM((_CB, _L, _D), jnp.float32),
            pltpu.VMEM((_CB, _L, _D), jnp.float32),
            pltpu.VMEM((_CB, _D), jnp.float32),
            pltpu.SemaphoreType.DMA,
            pltpu.SemaphoreType.DMA,
        ],
        compiler_params=pltpu.CompilerParams(use_tc_tiling_on_sc=False),
    )
    def body(text_hbm, emb_hbm, out_hbm, idx0, idx1, rows0, rows1, outb,
             sem0, sem1):
        wid = lax.axis_index("s") * _NC + lax.axis_index("c")
        base_row = wid * _ROWS_W

        idx_bufs = (idx0, idx1)
        row_bufs = (rows0, rows1)
        sems = (sem0, sem1)

        zero = jnp.zeros((16,), jnp.float32)

        def stage_and_fire(ch, b):
            pltpu.sync_copy(
                text_hbm.at[pl.ds(base_row + ch * _CB, _CB)], idx_bufs[b])

            def fire(i, _):
                pltpu.async_copy(emb_hbm.at[idx_bufs[b].at[i]],
                                 row_bufs[b].at[i], sems[b])
                return 0
            lax.fori_loop(0, _CB, fire, 0)

        stage_and_fire(0, 0)

        for ch in range(_NCH):
            cur = ch % 2
            nxt = (ch + 1) % 2
            if ch + 1 < _NCH:
                stage_and_fire(ch + 1, nxt)
            cur_rows = row_bufs[cur]
            cur_idx = idx_bufs[cur]
            cur_sem = sems[cur]

            def row_body(i, _):
                pltpu.make_async_copy(emb_hbm.at[cur_idx.at[i]],
                                      cur_rows.at[i], cur_sem).wait()

                def l_body(l, carry):
                    a0, a1 = carry
                    return (a0 + cur_rows[i, l, pl.ds(0, 16)],
                            a1 + cur_rows[i, l, pl.ds(16, 16)])
                a0, a1 = lax.fori_loop(0, _L, l_body, (zero, zero),
                                       unroll=10)
                outb[i, pl.ds(0, 16)] = a0
                outb[i, pl.ds(16, 16)] = a1
                return 0

            lax.fori_loop(0, _CB, row_body, 0)
            pltpu.sync_copy(outb, out_hbm.at[pl.ds(base_row + ch * _CB, _CB)])

    return body(text_flat, emb_weight)


def _linear_body(x_ref, w_ref, b_ref, o_ref):
    y = lax.dot_general(x_ref[...], w_ref[...], (((1,), (1,)), ((), ())),
                        preferred_element_type=jnp.float32)
    o_ref[...] = y * (1.0 / _L) + b_ref[...]


def _linear(bag, fc_weight, fc_bias2d):
    return pl.pallas_call(
        _linear_body,
        out_shape=jax.ShapeDtypeStruct((_B, _C), jnp.float32),
    )(bag, fc_weight, fc_bias2d)


def kernel(text, emb_weight, fc_weight, fc_bias):
    emb_lin = _relayout_table(emb_weight.T)
    emb_rows = emb_lin.reshape(_V, _D)
    bag = _embed_bag_sum(text, emb_rows)
    return _linear(bag, fc_weight, fc_bias.reshape(1, _C))


# speedup vs baseline: 1.2273x; 1.0163x over previous
import functools

import jax
import jax.numpy as jnp
from jax import lax
from jax.experimental import pallas as pl
from jax.experimental.pallas import tpu as pltpu
from jax.experimental.pallas import tpu_sc as plsc

_B = 16384
_L = 50
_D = 32
_C = 4
_NC = 2
_NS = 16
_NW = _NC * _NS
_ROWS_W = _B // _NW
_CB = 32
_NCH = _ROWS_W // _CB
_CHUNK_IDX = _CB * _L


_V = 1000000
_TBLK = 8192


def _transpose_body(xt_ref, o_ref):
    lane = lax.broadcasted_iota(jnp.int32, (_D, 128), 1)
    drow = lax.broadcasted_iota(jnp.int32, (_D, 128), 0)
    rep = jnp.where(lane % _D == drow, 1.0, 0.0).astype(jnp.float32)
    y = lax.dot_general(xt_ref[...], rep, (((0,), (0,)), ((), ())),
                        preferred_element_type=jnp.float32)
    sub = lax.broadcasted_iota(jnp.int32, (_TBLK, 128), 0)
    lane2 = lax.broadcasted_iota(jnp.int32, (_TBLK, 128), 1)
    ym = jnp.where(lane2 // _D == sub % 4, y, 0.0)
    o_ref[...] = ym.reshape(_TBLK // 4, 4, 128).sum(axis=1)


def _relayout_table(emb_t):
    grid = (_V + _TBLK - 1) // _TBLK
    return pl.pallas_call(
        _transpose_body,
        grid=(grid,),
        in_specs=[pl.BlockSpec((_D, _TBLK), lambda i: (0, i))],
        out_specs=pl.BlockSpec((_TBLK * _D // 128, 128), lambda i: (i, 0)),
        out_shape=jax.ShapeDtypeStruct((_V * _D // 128, 128), jnp.float32),
    )(emb_t)


def _embed_bag_sum(text_flat, emb_weight):
    mesh = plsc.VectorSubcoreMesh(core_axis_name="c", subcore_axis_name="s")

    @functools.partial(
        pl.kernel,
        out_type=jax.ShapeDtypeStruct((_B, _D), jnp.float32),
        mesh=mesh,
        scratch_types=[
            pltpu.VMEM((_CHUNK_IDX,), jnp.int32),
            pltpu.VMEM((_CHUNK_IDX,), jnp.int32),
            pltpu.VMEM((_CHUNK_IDX, _D), jnp.float32),
            pltpu.VMEM((_CHUNK_IDX, _D), jnp.float32),
            pltpu.VMEM((_CB, _D), jnp.float32),
            pltpu.SemaphoreType.DMA,
            pltpu.SemaphoreType.DMA,
        ],
        compiler_params=pltpu.CompilerParams(use_tc_tiling_on_sc=False),
    )
    def body(text_hbm, emb_hbm, out_hbm, idx0, idx1, rows0, rows1, outb,
             sem0, sem1):
        wid = lax.axis_index("s") * _NC + lax.axis_index("c")
        base_row = wid * _ROWS_W
        base_idx = base_row * _L

        idx_bufs = (idx0, idx1)
        row_bufs = (rows0, rows1)
        sems = (sem0, sem1)

        zero = jnp.zeros((16,), jnp.float32)

        def stage_and_fire(ch, b):
            pltpu.sync_copy(
                text_hbm.at[pl.ds(base_idx + ch * _CHUNK_IDX, _CHUNK_IDX)],
                idx_bufs[b])
            pltpu.async_copy(emb_hbm.at[idx_bufs[b]], row_bufs[b], sems[b])

        stage_and_fire(0, 0)

        for ch in range(_NCH):
            cur = ch % 2
            nxt = (ch + 1) % 2
            if ch + 1 < _NCH:
                stage_and_fire(ch + 1, nxt)
            cur_rows = row_bufs[cur]
            cur_idx = idx_bufs[cur]
            cur_sem = sems[cur]

            pltpu.make_async_copy(emb_hbm.at[cur_idx], cur_rows,
                                  cur_sem).wait()

            def row_body(i, _):
                def l_body(l, carry):
                    a0, a1 = carry
                    return (a0 + cur_rows[i * _L + l, pl.ds(0, 16)],
                            a1 + cur_rows[i * _L + l, pl.ds(16, 16)])
                a0, a1 = lax.fori_loop(0, _L, l_body, (zero, zero),
                                       unroll=10)
                outb[i, pl.ds(0, 16)] = a0
                outb[i, pl.ds(16, 16)] = a1
                return 0

            lax.fori_loop(0, _CB, row_body, 0)
            pltpu.sync_copy(outb, out_hbm.at[pl.ds(base_row + ch * _CB, _CB)])

    return body(text_flat, emb_weight)


def _linear_body(x_ref, w_ref, b_ref, o_ref):
    y = lax.dot_general(x_ref[...], w_ref[...], (((1,), (1,)), ((), ())),
                        preferred_element_type=jnp.float32)
    o_ref[...] = y * (1.0 / _L) + b_ref[...]


def _linear(bag, fc_weight, fc_bias2d):
    return pl.pallas_call(
        _linear_body,
        out_shape=jax.ShapeDtypeStruct((_B, _C), jnp.float32),
    )(bag, fc_weight, fc_bias2d)


def kernel(text, emb_weight, fc_weight, fc_bias):
    emb_lin = _relayout_table(emb_weight.T)
    emb_rows = emb_lin.reshape(_V, _D)
    bag = _embed_bag_sum(text.reshape(_B * _L), emb_rows)
    return _linear(bag, fc_weight, fc_bias.reshape(1, _C))
